# baseline (speedup 1.0000x reference)
import functools

import jax
import jax.numpy as jnp
from jax import lax
from jax.experimental import pallas as pl
from jax.experimental.pallas import tpu as pltpu
from jax.experimental.pallas import tpu_sc as plsc

NC, NS, L = 2, 16, 16
NW = NC * NS
TS = 10_000_000
MM = 1_000_000
DD = 32
BB = 262_144
PPW = BB // NW
SPW = MM // NW
WPAD = ((SPW + L - 1) // L) * L
CA = 2048
CB = 4096
CC = 512
CSH = 9
NCC = 64

VOXEL_INV = None
P_MOD = (73856093 % TS, 19349669 % TS, 83492791 % TS)

_mesh = plsc.VectorSubcoreMesh(core_axis_name="c", subcore_axis_name="s",
                               num_cores=NC, num_subcores=NS)


def _i32(v):
    return v if v.dtype == jnp.int32 else lax.convert_element_type(v, jnp.int32)


def _fori(hi, body, init, unroll=1):
    if hasattr(hi, 'dtype'):
        return lax.fori_loop(jnp.int32(0), hi, body, init)
    return lax.fori_loop(0, hi, body, init, unroll=unroll)


def _mulmod(g, p, tv):
    neg = g < 0
    a = jnp.abs(g)
    a = a - lax.div(a, tv) * tv
    r = jnp.zeros_like(a)
    for shift in (18, 12, 6, 0):
        d = (a >> shift) & 63
        r = r * 64 + d * p
        r = r - lax.div(r, tv) * tv
    r = jnp.where(neg & (r > 0), tv - r, r)
    return r


def _hash_body(points_ref, table_ref, slots_ref, pts_v, hash_v, low_v,
               sem):
    w = _i32(lax.axis_index("s")) * NC + _i32(lax.axis_index("c"))
    lane = lax.iota(jnp.int32, L)
    tv = jnp.full((L,), TS, jnp.int32)

    for c in range(PPW // CA):
        base = w * PPW + c * CA
        for d in range(3):
            pltpu.sync_copy(points_ref.at[d, pl.ds(base, CA)],
                            pts_v.at[d])

        def grp(g, _):
            gi = _i32(g)
            off = gi * L
            x = pts_v[0, pl.ds(off, L)]
            y = pts_v[1, pl.ds(off, L)]
            z = pts_v[2, pl.ds(off, L)]
            h = jnp.zeros((L,), jnp.int32)
            for coord, pm in zip((x, y, z), P_MOD):
                q = coord / jnp.float32(0.3)
                gc = q.astype(jnp.int32)
                gc = jnp.where(gc.astype(jnp.float32) > q, gc - 1, gc)
                h = h + _mulmod(gc, pm, tv)
            h = h - lax.div(h, tv) * tv
            hash_v[pl.ds(off, L)] = h * 2
            return 0

        _fori(CA // L, grp, 0)
        pltpu.async_copy(table_ref.at[hash_v], low_v, sem).wait()
        pltpu.sync_copy(low_v, slots_ref.at[pl.ds(base, CA)])

    del lane, tv


def _scatter_body(values_ref, slots_ref, out_ref, win_v, sl_v0,
                  sl_v1, cidx_v, cslot_v, rows_v, sem_b0, sem_b1,
                  sem_mv):
    w = _i32(lax.axis_index("s")) * NC + _i32(lax.axis_index("c"))
    lo = w * SPW
    lane = lax.iota(jnp.int32, L)

    neg1 = jnp.full((L,), -1, jnp.int32)

    def initg(g, _):
        gi = _i32(g)
        win_v[pl.ds(gi * L, L)] = neg1
        return 0

    _fori(WPAD // L, initg, 0, unroll=8)

    uspw = jnp.uint32(SPW)
    nch = BB // CB
    bufs = (sl_v0, sl_v1)
    sems = (sem_b0, sem_b1)
    cps = [pltpu.async_copy(slots_ref.at[pl.ds(0, CB)], sl_v0, sem_b0), None]
    for c in range(nch):
        b = c & 1
        if c + 1 < nch:
            cps[1 - b] = pltpu.async_copy(
                slots_ref.at[pl.ds((c + 1) * CB, CB)], bufs[1 - b],
                sems[1 - b])
        cps[b].wait()
        buf = bufs[b]

        def grp(g, _):
            gi = _i32(g)
            local = buf[pl.ds(gi * L, L)] - lo
            m = plsc.bitcast(local, jnp.uint32) < uspw
            iv = (c * CB + gi * L) + lane
            plsc.store_scatter(win_v, [local], iv, mask=m)
            return 0

        _fori(CB // L, grp, 0, unroll=8)

    def compg(g, acc):
        gi = _i32(g)
        wv = win_v[pl.ds(gi * L, L)]
        m = wv >= 0
        pos = acc + plsc.cumsum(m.astype(jnp.int32)) - 1
        slotv = lo + gi * L + lane
        plsc.store_scatter(cidx_v, [pos >> CSH, pos & (CC - 1)], wv, mask=m)
        plsc.store_scatter(cslot_v, [pos >> CSH, pos & (CC - 1)], slotv,
                           mask=m)
        return acc + plsc.all_reduce_population_count(m)

    acc = _fori(WPAD // L, compg, jnp.zeros((L,), jnp.int32))
    n = lax.reduce_max_p.bind(acc, axes=(0,))
    nfull = lax.div(n + jnp.int32(CC - 1), jnp.int32(CC))

    zero = jnp.zeros((L,), jnp.int32)
    fid = plsc.load_gather(cidx_v, [zero, zero])
    fsl = plsc.load_gather(cslot_v, [zero, zero])

    def fillg(g, _):
        gi = _i32(g)
        pos = n + gi * L + lane
        m = pos < nfull * CC
        posc = jnp.where(m, pos, 0)
        plsc.store_scatter(cidx_v, [posc >> CSH, posc & (CC - 1)], fid,
                           mask=m)
        plsc.store_scatter(cslot_v, [posc >> CSH, posc & (CC - 1)], fsl,
                           mask=m)
        return 0

    _fori(CC // L, fillg, 0, unroll=4)

    def mv(t, _):
        ti = _i32(t)
        pltpu.async_copy(values_ref.at[cidx_v.at[ti]], rows_v,
                         sem_mv).wait()
        pltpu.async_copy(rows_v, out_ref.at[cslot_v.at[ti]],
                         sem_mv).wait()
        return 0

    _fori(nfull, mv, 0)


@functools.partial(
    pl.kernel,
    out_type=jax.ShapeDtypeStruct((BB,), jnp.int32),
    mesh=_mesh,
    compiler_params=pltpu.CompilerParams(needs_layout_passes=False, use_tc_tiling_on_sc=False),
    scratch_types=[
        pltpu.VMEM((3, CA), jnp.float32),
        pltpu.VMEM((CA,), jnp.int32),
        pltpu.VMEM((CA,), jnp.int32),
        pltpu.SemaphoreType.DMA,
    ],
)
def _hash_kernel(points_ref, table_ref, slots_ref, pts_v, hash_v, low_v,
                 sem):
    _hash_body(points_ref, table_ref, slots_ref, pts_v, hash_v, low_v, sem)


@functools.partial(
    pl.kernel,
    out_type=(),
    mesh=_mesh,
    compiler_params=pltpu.CompilerParams(needs_layout_passes=False, use_tc_tiling_on_sc=False),
    scratch_types=[
        pltpu.VMEM((WPAD,), jnp.int32),
        pltpu.VMEM((CB,), jnp.int32),
        pltpu.VMEM((CB,), jnp.int32),
        pltpu.VMEM((NCC, CC), jnp.int32),
        pltpu.VMEM((NCC, CC), jnp.int32),
        pltpu.VMEM((CC, DD), jnp.float32),
        pltpu.SemaphoreType.DMA,
        pltpu.SemaphoreType.DMA,
        pltpu.SemaphoreType.DMA,
    ],
)
def _scatter_kernel(values_ref, slots_ref, out_ref, win_v, sl_v0,
                    sl_v1, cidx_v, cslot_v, rows_v, sem_b0,
                    sem_b1, sem_mv):
    _scatter_body(values_ref, slots_ref, out_ref, win_v, sl_v0,
                  sl_v1, cidx_v, cslot_v, rows_v, sem_b0,
                  sem_b1, sem_mv)


def kernel(mem, points, values, buffer_pt_index):
    pts_t = points.T
    table_flat = lax.bitcast_convert_type(buffer_pt_index,
                                          jnp.int32).reshape(-1)
    with jax.enable_x64(False):
        slots = _hash_kernel(pts_t, table_flat)
        out_ref = jax.new_ref(mem)
        _scatter_kernel(values, slots, out_ref)
        out = out_ref[...]
    return out


# speedup vs baseline: 9.9071x; 9.9071x over previous
import functools

import jax
import jax.numpy as jnp
from jax import lax
from jax.experimental import pallas as pl
from jax.experimental.pallas import tpu as pltpu
from jax.experimental.pallas import tpu_sc as plsc

NC, NS, L = 2, 16, 16
NW = NC * NS
TS = 10_000_000
MM = 1_000_000
DD = 32
BB = 262_144
PPW = BB // NW
SPW = MM // NW
WPAD = ((SPW + L - 1) // L) * L
CA = 2048
CB = 4096
CC = 512
CSH = 9
NCC = 64

VOXEL_INV = None
P_MOD = (73856093 % TS, 19349669 % TS, 83492791 % TS)

_mesh = plsc.VectorSubcoreMesh(core_axis_name="c", subcore_axis_name="s",
                               num_cores=NC, num_subcores=NS)


def _i32(v):
    return v if v.dtype == jnp.int32 else lax.convert_element_type(v, jnp.int32)


def _fori(hi, body, init, unroll=1):
    if hasattr(hi, 'dtype'):
        return lax.fori_loop(jnp.int32(0), hi, body, init)
    return lax.fori_loop(0, hi, body, init, unroll=unroll)


def _mulmod(g, p, tv):
    neg = g < 0
    a = jnp.abs(g)
    a = a - lax.div(a, tv) * tv
    r = jnp.zeros_like(a)
    for shift in (18, 12, 6, 0):
        d = (a >> shift) & 63
        r = r * 64 + d * p
        r = r - lax.div(r, tv) * tv
    r = jnp.where(neg & (r > 0), tv - r, r)
    return r


def _hash_body(points_ref, table_ref, slots_ref, pts_v, hash_v, low_v,
               sem):
    w = _i32(lax.axis_index("s")) * NC + _i32(lax.axis_index("c"))
    lane = lax.iota(jnp.int32, L)
    tv = jnp.full((L,), TS, jnp.int32)

    for c in range(PPW // CA):
        base = w * PPW + c * CA
        for d in range(3):
            pltpu.sync_copy(points_ref.at[d, pl.ds(base, CA)],
                            pts_v.at[d])

        def grp(g, _):
            gi = _i32(g)
            off = gi * L
            x = pts_v[0, pl.ds(off, L)]
            y = pts_v[1, pl.ds(off, L)]
            z = pts_v[2, pl.ds(off, L)]
            h = jnp.zeros((L,), jnp.int32)
            for coord, pm in zip((x, y, z), P_MOD):
                q = coord / jnp.float32(0.3)
                gc = q.astype(jnp.int32)
                gc = jnp.where(gc.astype(jnp.float32) > q, gc - 1, gc)
                h = h + _mulmod(gc, pm, tv)
            h = h - lax.div(h, tv) * tv
            hash_v[pl.ds(off, L)] = h
            return 0

        _fori(CA // L, grp, 0)
        pltpu.async_copy(table_ref.at[hash_v], low_v, sem).wait()
        pltpu.sync_copy(low_v, slots_ref.at[pl.ds(base, CA)])

    del lane, tv


def _scatter_body(values_ref, slots_ref, out_ref, win_v, sl_v0,
                  sl_v1, cidx_v, cslot_v, rows_v, sem_b0, sem_b1,
                  sem_mv):
    w = _i32(lax.axis_index("s")) * NC + _i32(lax.axis_index("c"))
    lo = w * SPW
    lane = lax.iota(jnp.int32, L)

    neg1 = jnp.full((L,), -1, jnp.int32)

    def initg(g, _):
        gi = _i32(g)
        win_v[pl.ds(gi * L, L)] = neg1
        return 0

    _fori(WPAD // L, initg, 0, unroll=8)

    uspw = jnp.uint32(SPW)
    nch = BB // CB
    bufs = (sl_v0, sl_v1)
    sems = (sem_b0, sem_b1)
    cps = [pltpu.async_copy(slots_ref.at[pl.ds(0, CB)], sl_v0, sem_b0), None]
    for c in range(nch):
        b = c & 1
        if c + 1 < nch:
            cps[1 - b] = pltpu.async_copy(
                slots_ref.at[pl.ds((c + 1) * CB, CB)], bufs[1 - b],
                sems[1 - b])
        cps[b].wait()
        buf = bufs[b]

        def grp(g, _):
            gi = _i32(g)
            local = buf[pl.ds(gi * L, L)] - lo
            m = plsc.bitcast(local, jnp.uint32) < uspw
            iv = (c * CB + gi * L) + lane
            plsc.store_scatter(win_v, [local], iv, mask=m)
            return 0

        _fori(CB // L, grp, 0, unroll=8)

    def compg(g, acc):
        gi = _i32(g)
        wv = win_v[pl.ds(gi * L, L)]
        m = wv >= 0
        pos = acc + plsc.cumsum(m.astype(jnp.int32)) - 1
        slotv = lo + gi * L + lane
        plsc.store_scatter(cidx_v, [pos >> CSH, pos & (CC - 1)], wv, mask=m)
        plsc.store_scatter(cslot_v, [pos >> CSH, pos & (CC - 1)], slotv,
                           mask=m)
        return acc + plsc.all_reduce_population_count(m)

    acc = _fori(WPAD // L, compg, jnp.zeros((L,), jnp.int32))
    n = lax.reduce_max_p.bind(acc, axes=(0,))
    nfull = lax.div(n + jnp.int32(CC - 1), jnp.int32(CC))

    zero = jnp.zeros((L,), jnp.int32)
    fid = plsc.load_gather(cidx_v, [zero, zero])
    fsl = plsc.load_gather(cslot_v, [zero, zero])

    def fillg(g, _):
        gi = _i32(g)
        pos = n + gi * L + lane
        m = pos < nfull * CC
        posc = jnp.where(m, pos, 0)
        plsc.store_scatter(cidx_v, [posc >> CSH, posc & (CC - 1)], fid,
                           mask=m)
        plsc.store_scatter(cslot_v, [posc >> CSH, posc & (CC - 1)], fsl,
                           mask=m)
        return 0

    _fori(CC // L, fillg, 0, unroll=4)

    def mv(t, _):
        ti = _i32(t)
        pltpu.async_copy(values_ref.at[cidx_v.at[ti]], rows_v,
                         sem_mv).wait()
        pltpu.async_copy(rows_v, out_ref.at[cslot_v.at[ti]],
                         sem_mv).wait()
        return 0

    _fori(nfull, mv, 0)


@functools.partial(
    pl.kernel,
    out_type=jax.ShapeDtypeStruct((BB,), jnp.int32),
    mesh=_mesh,
    compiler_params=pltpu.CompilerParams(needs_layout_passes=False, use_tc_tiling_on_sc=False),
    scratch_types=[
        pltpu.VMEM((3, CA), jnp.float32),
        pltpu.VMEM((CA,), jnp.int32),
        pltpu.VMEM((CA,), jnp.int32),
        pltpu.SemaphoreType.DMA,
    ],
)
def _hash_kernel(points_ref, table_ref, slots_ref, pts_v, hash_v, low_v,
                 sem):
    _hash_body(points_ref, table_ref, slots_ref, pts_v, hash_v, low_v, sem)


@functools.partial(
    pl.kernel,
    out_type=(),
    mesh=_mesh,
    compiler_params=pltpu.CompilerParams(needs_layout_passes=False, use_tc_tiling_on_sc=False),
    scratch_types=[
        pltpu.VMEM((WPAD,), jnp.int32),
        pltpu.VMEM((CB,), jnp.int32),
        pltpu.VMEM((CB,), jnp.int32),
        pltpu.VMEM((NCC, CC), jnp.int32),
        pltpu.VMEM((NCC, CC), jnp.int32),
        pltpu.VMEM((CC, DD), jnp.float32),
        pltpu.SemaphoreType.DMA,
        pltpu.SemaphoreType.DMA,
        pltpu.SemaphoreType.DMA,
    ],
)
def _scatter_kernel(values_ref, slots_ref, out_ref, win_v, sl_v0,
                    sl_v1, cidx_v, cslot_v, rows_v, sem_b0,
                    sem_b1, sem_mv):
    _scatter_body(values_ref, slots_ref, out_ref, win_v, sl_v0,
                  sl_v1, cidx_v, cslot_v, rows_v, sem_b0,
                  sem_b1, sem_mv)


def kernel(mem, points, values, buffer_pt_index):
    pts_t = points.T
    table1 = buffer_pt_index.astype(jnp.int32)
    with jax.enable_x64(False):
        slots = _hash_kernel(pts_t, table1)
        out_ref = jax.new_ref(mem)
        _scatter_kernel(values, slots, out_ref)
        out = out_ref[...]
    return out


# speedup vs baseline: 9.9151x; 1.0008x over previous
import functools

import jax
import jax.numpy as jnp
from jax import lax
from jax.experimental import pallas as pl
from jax.experimental.pallas import tpu as pltpu
from jax.experimental.pallas import tpu_sc as plsc

NC, NS, L = 2, 16, 16
NW = NC * NS
TS = 10_000_000
MM = 1_000_000
DD = 32
BB = 262_144
PPW = BB // NW
SPW = MM // NW
WPAD = ((SPW + L - 1) // L) * L
CA = 2048
CB = 4096
CC = 512
CSH = 9
NCC = 64

VOXEL_INV = None
P_MOD = (73856093 % TS, 19349669 % TS, 83492791 % TS)

_mesh = plsc.VectorSubcoreMesh(core_axis_name="c", subcore_axis_name="s",
                               num_cores=NC, num_subcores=NS)


def _i32(v):
    return v if v.dtype == jnp.int32 else lax.convert_element_type(v, jnp.int32)


def _fori(hi, body, init, unroll=1):
    if hasattr(hi, 'dtype'):
        return lax.fori_loop(jnp.int32(0), hi, body, init)
    return lax.fori_loop(0, hi, body, init, unroll=unroll)


def _mulmod(g, p, tv):
    neg = g < 0
    a = jnp.abs(g)
    a = a - lax.div(a, tv) * tv
    r = jnp.zeros_like(a)
    for shift in (18, 12, 6, 0):
        d = (a >> shift) & 63
        r = r * 64 + d * p
        r = r - lax.div(r, tv) * tv
    r = jnp.where(neg & (r > 0), tv - r, r)
    return r


def _hash_body(points_ref, table_ref, slots_ref, pts_v, hash_v, low_v,
               sem):
    w = _i32(lax.axis_index("s")) * NC + _i32(lax.axis_index("c"))
    lane = lax.iota(jnp.int32, L)
    tv = jnp.full((L,), TS, jnp.int32)

    for c in range(PPW // CA):
        base = w * PPW + c * CA
        for d in range(3):
            pltpu.sync_copy(points_ref.at[d, pl.ds(base, CA)],
                            pts_v.at[d])

        def grp(g, _):
            gi = _i32(g)
            off = gi * L
            x = pts_v[0, pl.ds(off, L)]
            y = pts_v[1, pl.ds(off, L)]
            z = pts_v[2, pl.ds(off, L)]
            h = jnp.zeros((L,), jnp.int32)
            for coord, pm in zip((x, y, z), P_MOD):
                q = coord / jnp.float32(0.3)
                gc = q.astype(jnp.int32)
                gc = jnp.where(gc.astype(jnp.float32) > q, gc - 1, gc)
                h = h + _mulmod(gc, pm, tv)
            h = h - lax.div(h, tv) * tv
            hash_v[pl.ds(off, L)] = h
            return 0

        _fori(CA // L, grp, 0)
        pltpu.async_copy(table_ref.at[hash_v], low_v, sem).wait()
        pltpu.sync_copy(low_v, slots_ref.at[pl.ds(base, CA)])

    del lane, tv


def _scatter_body(values_ref, slots_ref, out_ref, win_v, sl_v0,
                  sl_v1, cidx_v, cslot_v, rows_v, sem_b0, sem_b1,
                  sem_mv):
    w = _i32(lax.axis_index("s")) * NC + _i32(lax.axis_index("c"))
    lo = w * SPW
    lane = lax.iota(jnp.int32, L)

    neg1 = jnp.full((L,), -1, jnp.int32)

    def initg(g, _):
        gi = _i32(g)
        win_v[pl.ds(gi * L, L)] = neg1
        return 0

    _fori(WPAD // L, initg, 0, unroll=8)

    uspw = jnp.uint32(SPW)
    nch = BB // CB
    bufs = (sl_v0, sl_v1)
    sems = (sem_b0, sem_b1)
    cps = [pltpu.async_copy(slots_ref.at[pl.ds(0, CB)], sl_v0, sem_b0), None]
    for c in range(nch):
        b = c & 1
        if c + 1 < nch:
            cps[1 - b] = pltpu.async_copy(
                slots_ref.at[pl.ds((c + 1) * CB, CB)], bufs[1 - b],
                sems[1 - b])
        cps[b].wait()
        buf = bufs[b]

        def grp(g, _):
            gi = _i32(g)
            local = buf[pl.ds(gi * L, L)] - lo
            m = plsc.bitcast(local, jnp.uint32) < uspw
            iv = (c * CB + gi * L) + lane
            plsc.store_scatter(win_v, [local], iv, mask=m)
            return 0

        _fori(CB // L, grp, 0, unroll=8)

    def compg(g, acc):
        gi = _i32(g)
        wv = win_v[pl.ds(gi * L, L)]
        m = wv >= 0
        pos = acc + plsc.cumsum(m.astype(jnp.int32)) - 1
        slotv = lo + gi * L + lane
        plsc.store_scatter(cidx_v, [pos >> CSH, pos & (CC - 1)], wv, mask=m)
        plsc.store_scatter(cslot_v, [pos >> CSH, pos & (CC - 1)], slotv,
                           mask=m)
        return acc + plsc.all_reduce_population_count(m)

    acc = _fori(WPAD // L, compg, jnp.zeros((L,), jnp.int32))
    n = lax.reduce_max_p.bind(acc, axes=(0,))
    nfull = lax.div(n + jnp.int32(CC - 1), jnp.int32(CC))

    zero = jnp.zeros((L,), jnp.int32)
    fid = plsc.load_gather(cidx_v, [zero, zero])
    fsl = plsc.load_gather(cslot_v, [zero, zero])

    def fillg(g, _):
        gi = _i32(g)
        pos = n + gi * L + lane
        m = pos < nfull * CC
        posc = jnp.where(m, pos, 0)
        plsc.store_scatter(cidx_v, [posc >> CSH, posc & (CC - 1)], fid,
                           mask=m)
        plsc.store_scatter(cslot_v, [posc >> CSH, posc & (CC - 1)], fsl,
                           mask=m)
        return 0

    _fori(CC // L, fillg, 0, unroll=4)

    def mv(t, _):
        ti = _i32(t)
        pltpu.async_copy(values_ref.at[cidx_v.at[ti]], rows_v,
                         sem_mv).wait()
        pltpu.async_copy(rows_v, out_ref.at[cslot_v.at[ti]],
                         sem_mv).wait()
        return 0

    _fori(nfull, mv, 0)


@functools.partial(
    pl.kernel,
    out_type=jax.ShapeDtypeStruct((BB,), jnp.int32),
    mesh=_mesh,
    compiler_params=pltpu.CompilerParams(needs_layout_passes=False, use_tc_tiling_on_sc=False),
    scratch_types=[
        pltpu.VMEM((3, CA), jnp.float32),
        pltpu.VMEM((CA,), jnp.int32),
        pltpu.VMEM((CA,), jnp.int32),
        pltpu.SemaphoreType.DMA,
    ],
)
def _hash_kernel(points_ref, table_ref, slots_ref, pts_v, hash_v, low_v,
                 sem):
    _hash_body(points_ref, table_ref, slots_ref, pts_v, hash_v, low_v, sem)


@functools.partial(
    pl.kernel,
    out_type=(),
    mesh=_mesh,
    compiler_params=pltpu.CompilerParams(needs_layout_passes=False, use_tc_tiling_on_sc=False),
    scratch_types=[
        pltpu.VMEM((WPAD,), jnp.int32),
        pltpu.VMEM((CB,), jnp.int32),
        pltpu.VMEM((CB,), jnp.int32),
        pltpu.VMEM((NCC, CC), jnp.int32),
        pltpu.VMEM((NCC, CC), jnp.int32),
        pltpu.VMEM((CC, DD), jnp.float32),
        pltpu.SemaphoreType.DMA,
        pltpu.SemaphoreType.DMA,
        pltpu.SemaphoreType.DMA,
    ],
)
def _scatter_kernel(values_ref, slots_ref, out_ref, win_v, sl_v0,
                    sl_v1, cidx_v, cslot_v, rows_v, sem_b0,
                    sem_b1, sem_mv):
    _scatter_body(values_ref, slots_ref, out_ref, win_v, sl_v0,
                  sl_v1, cidx_v, cslot_v, rows_v, sem_b0,
                  sem_b1, sem_mv)


def kernel(mem, points, values, buffer_pt_index):
    pts_t = points.T
    table1 = lax.bitcast_convert_type(buffer_pt_index.astype(jnp.uint32),
                                      jnp.int32)
    with jax.enable_x64(False):
        slots = _hash_kernel(pts_t, table1)
        out_ref = jax.new_ref(mem)
        _scatter_kernel(values, slots, out_ref)
        out = out_ref[...]
    return out
